# baseline (speedup 1.0000x reference)
import functools

import jax
import jax.numpy as jnp
from jax import lax
from jax.experimental import pallas as pl
from jax.experimental.pallas import tpu as pltpu
from jax.experimental.pallas import tpu_sc as plsc

_IN = 128
_N0 = 10000
_N1 = 5000
_N2 = 1024
_E0 = 320000
_E1 = 160000

_NC = 2
_NS = 16
_NW = _NC * _NS
_CB = 256


def _pad_edges(e):
    q = _NW * _CB * 2
    return ((e + q - 1) // q) * q


def _n_acc(n_tgt):
    return ((n_tgt + 1 + _NS * 8 - 1) // (_NS * 8)) * (_NS * 8)


def _make_agg(n_tgt, e_pad, n_tab, stage_tab):
    n_acc = _n_acc(n_tgt)
    rps = n_acc // _NS
    e_per_w = e_pad // _NW
    n_chunks = e_per_w // _CB
    n_pairs = n_chunks // 2
    tps = ((n_tab // _NS + 7) // 8) * 8

    mesh = plsc.VectorSubcoreMesh(core_axis_name="c", subcore_axis_name="s")

    @functools.partial(
        pl.kernel,
        out_type=(
            jax.ShapeDtypeStruct((_NC, n_acc, _IN), jnp.float32),
            jax.ShapeDtypeStruct((_NW, n_acc), jnp.float32),
        ),
        mesh=mesh,
        scratch_types=[
            pltpu.VMEM_SHARED((n_acc, _IN), jnp.float32),
            pltpu.VMEM_SHARED((n_tab if stage_tab else 8, _IN), jnp.float32),
            pltpu.VMEM((n_acc,), jnp.float32),
            pltpu.VMEM((_CB,), jnp.int32),
            pltpu.VMEM((_CB,), jnp.int32),
            pltpu.VMEM((_CB,), jnp.int32),
            pltpu.VMEM((_CB,), jnp.int32),
            pltpu.VMEM((2, _CB, _IN), jnp.float32),
            pltpu.SemaphoreType.DMA,
            pltpu.SemaphoreType.DMA,
            pltpu.SemaphoreType.DMA,
            pltpu.SemaphoreType.DMA,
        ],
        compiler_params=pltpu.CompilerParams(needs_layout_passes=False),
    )
    def agg(tab, pk, zsum, zcnt, osum, ocnt,
            acc, tabs, cnt_v, src0, src1, tgt0, tgt1, rows2,
            semg0, semg1, semi0, semi1):
        cid = lax.axis_index("c")
        sid = lax.axis_index("s")
        wid = cid * _NS + sid
        r0 = sid * rps
        rowbase = wid * n_chunks
        pltpu.sync_copy(zsum.at[pl.ds(r0, rps)], acc.at[pl.ds(r0, rps)])
        pltpu.sync_copy(zcnt, cnt_v)
        if stage_tab:
            t0 = sid * tps

            @pl.when(sid < _NS - 1)
            def _():
                pltpu.sync_copy(tab.at[pl.ds(t0, tps)],
                                tabs.at[pl.ds(t0, tps)])

            last = n_tab - (_NS - 1) * tps

            @pl.when(sid == _NS - 1)
            def _():
                pltpu.sync_copy(tab.at[pl.ds((_NS - 1) * tps, last)],
                                tabs.at[pl.ds((_NS - 1) * tps, last)])

        plsc.subcore_barrier()
        gtab = tabs if stage_tab else tab

        ones16 = jnp.ones((16,), jnp.float32)
        srcs = [src0, src1]
        tgts = [tgt0, tgt1]
        semg = [semg0, semg1]
        semi = [semi0, semi1]

        def fire_idx(c, p):
            row = rowbase + c
            pltpu.async_copy(pk.at[row, pl.ds(0, _CB)], srcs[p], semi[p])
            pltpu.async_copy(pk.at[row, pl.ds(_CB, _CB)], tgts[p], semi[p])

        def wait_idx(p):
            pltpu.make_async_copy(pk.at[0, pl.ds(0, _CB)], srcs[p],
                                  semi[p]).wait()
            pltpu.make_async_copy(pk.at[0, pl.ds(_CB, _CB)], tgts[p],
                                  semi[p]).wait()

        def fire_gather(p):
            pltpu.async_copy(gtab.at[srcs[p]], rows2.at[p], semg[p])

        def wait_gather(p):
            pltpu.make_async_copy(gtab.at[srcs[p]], rows2.at[p],
                                  semg[p]).wait()

        def consume(p):
            pltpu.sync_copy(rows2.at[p], acc.at[tgts[p]], add=True)
            for k in range(_CB // 16):
                plsc.addupdate_scatter(cnt_v, [tgts[p][pl.ds(k * 16, 16)]],
                                       ones16)

        fire_idx(0, 0)
        wait_idx(0)
        fire_gather(0)
        fire_idx(1, 1)

        def pair(j, carry):
            c0 = 2 * j
            wait_gather(0)

            @pl.when(j + 1 < n_pairs)
            def _():
                fire_idx(c0 + 2, 0)

            wait_idx(1)
            fire_gather(1)
            consume(0)

            wait_gather(1)

            @pl.when(j + 1 < n_pairs)
            def _():
                fire_idx(c0 + 3, 1)
                wait_idx(0)
                fire_gather(0)

            consume(1)
            return carry

        lax.fori_loop(0, n_pairs, pair, 0)
        plsc.subcore_barrier()
        pltpu.sync_copy(acc.at[pl.ds(r0, rps)], osum.at[cid, pl.ds(r0, rps)])
        pltpu.sync_copy(cnt_v, ocnt.at[wid])

    return agg


def _dense0(ps, pc, x, wl, wr, b2):
    blk = 1000

    def body(ps_ref, pc_ref, xt_ref, wl_ref, wr_ref, b_ref, o_ref):
        s = ps_ref[0] + ps_ref[1]
        c = jnp.sum(pc_ref[...], axis=1, keepdims=True)
        mean = s / jnp.maximum(c, 1.0)
        h = (jnp.dot(mean, wl_ref[...], preferred_element_type=jnp.float32)
             + jnp.dot(xt_ref[...], wr_ref[...], preferred_element_type=jnp.float32)
             + b_ref[...])
        o_ref[...] = jnp.maximum(h, 0.0)

    return pl.pallas_call(
        body,
        grid=(_N1 // blk,),
        in_specs=[
            pl.BlockSpec((_NC, blk, _IN), lambda i: (0, i, 0)),
            pl.BlockSpec((blk, _NW), lambda i: (i, 0)),
            pl.BlockSpec((blk, _IN), lambda i: (i, 0)),
            pl.BlockSpec((_IN, _IN), lambda i: (0, 0)),
            pl.BlockSpec((_IN, _IN), lambda i: (0, 0)),
            pl.BlockSpec((1, _IN), lambda i: (0, 0)),
        ],
        out_specs=pl.BlockSpec((blk, _IN), lambda i: (i, 0)),
        out_shape=jax.ShapeDtypeStruct((_N1, _IN), jnp.float32),
    )(ps, pc, x, wl, wr, b2)


def _dense1(ps, pc, h, wl, wr, b2):
    def body(ps_ref, pc_ref, ht_ref, wl_ref, wr_ref, b_ref, o_ref):
        s = ps_ref[0] + ps_ref[1]
        c = jnp.sum(pc_ref[...], axis=1, keepdims=True)
        mean = s / jnp.maximum(c, 1.0)
        z = (jnp.dot(mean, wl_ref[...], preferred_element_type=jnp.float32)
             + jnp.dot(ht_ref[...], wr_ref[...], preferred_element_type=jnp.float32)
             + b_ref[...])
        m = jnp.max(z, axis=-1, keepdims=True)
        e = jnp.exp(z - m)
        o_ref[...] = z - m - jnp.log(jnp.sum(e, axis=-1, keepdims=True))

    return pl.pallas_call(
        body,
        grid=(1,),
        in_specs=[
            pl.BlockSpec((_NC, _N2, _IN), lambda i: (0, 0, 0)),
            pl.BlockSpec((_N2, _NW), lambda i: (0, 0)),
            pl.BlockSpec((_N2, _IN), lambda i: (0, 0)),
            pl.BlockSpec((_IN, _IN), lambda i: (0, 0)),
            pl.BlockSpec((_IN, _IN), lambda i: (0, 0)),
            pl.BlockSpec((1, _IN), lambda i: (0, 0)),
        ],
        out_specs=pl.BlockSpec((_N2, _IN), lambda i: (0, 0)),
        out_shape=jax.ShapeDtypeStruct((_N2, _IN), jnp.float32),
    )(ps, pc, h, wl, wr, b2)


def kernel(x, Wl0, Wr0, b0, Wl1, Wr1, b1, edge_index_0, edge_index_1,
           size_0, size_1):
    e0p = _pad_edges(_E0)
    e1p = _pad_edges(_E1)
    src0 = jnp.concatenate(
        [edge_index_0[0].astype(jnp.int32),
         jnp.zeros((e0p - _E0,), jnp.int32)])
    tgt0 = jnp.concatenate(
        [edge_index_0[1].astype(jnp.int32),
         jnp.full((e0p - _E0,), _N1, jnp.int32)])
    src1 = jnp.concatenate(
        [edge_index_1[0].astype(jnp.int32),
         jnp.zeros((e1p - _E1,), jnp.int32)])
    tgt1 = jnp.concatenate(
        [edge_index_1[1].astype(jnp.int32),
         jnp.full((e1p - _E1,), _N2, jnp.int32)])
    pk0 = jnp.concatenate([src0.reshape(-1, _CB), tgt0.reshape(-1, _CB)],
                          axis=1)
    pk1 = jnp.concatenate([src1.reshape(-1, _CB), tgt1.reshape(-1, _CB)],
                          axis=1)

    na0 = _n_acc(_N1)
    na1 = _n_acc(_N2)
    zs0 = jnp.zeros((na0, _IN), jnp.float32)
    zc0 = jnp.zeros((na0,), jnp.float32)
    zs1 = jnp.zeros((na1, _IN), jnp.float32)
    zc1 = jnp.zeros((na1,), jnp.float32)

    agg0 = _make_agg(_N1, e0p, _N0, stage_tab=False)
    ps0, pc0 = agg0(x, pk0, zs0, zc0)
    h = _dense0(ps0, pc0.T, x, Wl0, Wr0, b0.reshape(1, _IN))

    agg1 = _make_agg(_N2, e1p, _N1, stage_tab=True)
    ps1, pc1 = agg1(h, pk1, zs1, zc1)
    out = _dense1(ps1, pc1.T, h, Wl1, Wr1, b1.reshape(1, _IN))
    return out
